# baseline (speedup 1.0000x reference)
import functools

import jax
import jax.numpy as jnp
from jax import lax
from jax.experimental import pallas as pl
from jax.experimental.pallas import tpu as pltpu
from jax.experimental.pallas import tpu_sc as plsc

EMBED = 64
CHUNK = 100


def _sc_body(x_hbm, table_hbm, pos_hbm, out_hbm, idx_v, pos_v, buf, sem,
             *, num_cores, chunks):
    wid = lax.axis_index("s") * num_cores + lax.axis_index("c")
    pltpu.sync_copy(x_hbm.at[wid], idx_v)
    pltpu.sync_copy(pos_hbm, pos_v)

    def chunk_body(j, carry):
        pltpu.async_copy(table_hbm.at[idx_v.at[j]], buf, sem).wait()
        pbase = lax.rem(j, 2) * CHUNK

        def row_body(r, c2):
            for c in range(EMBED // 16):
                sl = pl.ds(c * 16, 16)
                buf[r, sl] = buf[r, sl] + pos_v[pbase + r, sl]
            return c2

        lax.fori_loop(0, CHUNK, row_body, 0, unroll=False)
        pltpu.sync_copy(buf, out_hbm.at[wid, j])
        return carry

    lax.fori_loop(0, chunks, chunk_body, 0, unroll=False)


def kernel(x, token_emb, pos_emb):
    B, T = x.shape
    info = plsc.get_sparse_core_info()
    nw = info.num_cores * info.num_subcores
    total = B * T
    per_w = total // nw
    chunks = per_w // CHUNK
    assert per_w % CHUNK == 0 and per_w % T == 0 and T == 2 * CHUNK

    x_r = x.astype(jnp.int32).reshape(nw, chunks, CHUNK)
    pos2d = pos_emb[0, :T, :]

    mesh = plsc.VectorSubcoreMesh(core_axis_name="c", subcore_axis_name="s")
    body = functools.partial(_sc_body, num_cores=info.num_cores, chunks=chunks)
    out = pl.kernel(
        body,
        out_type=jax.ShapeDtypeStruct((nw, chunks, CHUNK, EMBED), jnp.float32),
        mesh=mesh,
        compiler_params=pltpu.CompilerParams(use_tc_tiling_on_sc=False),
        scratch_types=[
            pltpu.VMEM((chunks, CHUNK), jnp.int32),
            pltpu.VMEM((T, EMBED), jnp.float32),
            pltpu.VMEM((CHUNK, EMBED), jnp.float32),
            pltpu.SemaphoreType.DMA,
        ],
    )(x_r, token_emb, pos2d)
    return out.reshape(B, T, EMBED)


# speedup vs baseline: 1.3153x; 1.3153x over previous
import functools

import jax
import jax.numpy as jnp
from jax import lax
from jax.experimental import pallas as pl
from jax.experimental.pallas import tpu as pltpu
from jax.experimental.pallas import tpu_sc as plsc

EMBED = 64
CHUNK = 100


def _sc_body(x_hbm, table_hbm, pos_hbm, out_hbm, idx_v, pos_sh, buf, sem,
             *, num_cores, chunks):
    sid = lax.axis_index("s")
    wid = sid * num_cores + lax.axis_index("c")
    pltpu.sync_copy(x_hbm.at[wid], idx_v)

    @pl.when(sid == 0)
    def _():
        for h in range(2):
            sl = pl.ds(h * CHUNK, CHUNK)
            pltpu.sync_copy(pos_hbm.at[sl], buf)
            pltpu.sync_copy(buf, pos_sh.at[sl])

    plsc.subcore_barrier()

    def chunk_body(j, carry):
        pbase = lax.rem(j, 2) * CHUNK
        pltpu.sync_copy(pos_sh.at[pl.ds(pbase, CHUNK)], buf)
        pltpu.async_copy(table_hbm.at[idx_v.at[j]], buf, sem, add=True).wait()
        pltpu.sync_copy(buf, out_hbm.at[wid, j])
        return carry

    lax.fori_loop(0, chunks, chunk_body, 0, unroll=False)


def kernel(x, token_emb, pos_emb):
    B, T = x.shape
    info = plsc.get_sparse_core_info()
    nw = info.num_cores * info.num_subcores
    total = B * T
    per_w = total // nw
    chunks = per_w // CHUNK
    assert per_w % CHUNK == 0 and per_w % T == 0 and T == 2 * CHUNK

    x_r = x.astype(jnp.int32).reshape(nw, chunks, CHUNK)
    pos2d = pos_emb[0, :T, :]

    mesh = plsc.VectorSubcoreMesh(core_axis_name="c", subcore_axis_name="s")
    body = functools.partial(_sc_body, num_cores=info.num_cores, chunks=chunks)
    out = pl.kernel(
        body,
        out_type=jax.ShapeDtypeStruct((nw, chunks, CHUNK, EMBED), jnp.float32),
        mesh=mesh,
        compiler_params=pltpu.CompilerParams(use_tc_tiling_on_sc=False),
        scratch_types=[
            pltpu.VMEM((chunks, CHUNK), jnp.int32),
            pltpu.VMEM_SHARED((T, EMBED), jnp.float32),
            pltpu.VMEM((CHUNK, EMBED), jnp.float32),
            pltpu.SemaphoreType.DMA,
        ],
    )(x_r, token_emb, pos2d)
    return out.reshape(B, T, EMBED)


# speedup vs baseline: 1.5730x; 1.1960x over previous
import functools

import jax
import jax.numpy as jnp
from jax import lax
from jax.experimental import pallas as pl
from jax.experimental.pallas import tpu as pltpu
from jax.experimental.pallas import tpu_sc as plsc

EMBED = 64
CHUNK = 100
NBUF = 6
LOOK = 2


def _sc_body(x_hbm, table_hbm, pos_hbm, out_hbm, idx_v, pos_sh, bufs,
             psem, gsem, osem, *, num_cores, chunks):
    sid = lax.axis_index("s")
    wid = sid * num_cores + lax.axis_index("c")
    pltpu.sync_copy(x_hbm.at[wid], idx_v)

    @pl.when(sid == 0)
    def _():
        for h in range(2):
            sl = pl.ds(h * CHUNK, CHUNK)
            pltpu.sync_copy(pos_hbm.at[sl], bufs.at[0])
            pltpu.sync_copy(bufs.at[0], pos_sh.at[sl])

    plsc.subcore_barrier()

    def step(i, carry):
        b = lax.rem(i, NBUF)
        @pl.when(i >= NBUF)
        def _():
            pltpu.make_async_copy(
                bufs.at[b], out_hbm.at[wid, 0], osem.at[b]).wait()

        @pl.when(i < chunks)
        def _():
            pbase = lax.rem(i, 2) * CHUNK
            pltpu.async_copy(
                pos_sh.at[pl.ds(pbase, CHUNK)], bufs.at[b], psem.at[b])

        @pl.when(jnp.logical_and(i >= LOOK, i < chunks + LOOK))
        def _():
            j = i - LOOK
            bj = lax.rem(j, NBUF)
            pltpu.make_async_copy(
                pos_sh.at[pl.ds(0, CHUNK)], bufs.at[bj], psem.at[bj]).wait()
            pltpu.async_copy(
                table_hbm.at[idx_v.at[j]], bufs.at[bj], gsem.at[bj],
                add=True)

        @pl.when(jnp.logical_and(i >= 2 * LOOK, i < chunks + 2 * LOOK))
        def _():
            j = i - 2 * LOOK
            bj = lax.rem(j, NBUF)
            pltpu.make_async_copy(
                table_hbm.at[idx_v.at[j]], bufs.at[bj], gsem.at[bj]).wait()
            pltpu.async_copy(bufs.at[bj], out_hbm.at[wid, j], osem.at[bj])

        return carry

    lax.fori_loop(0, chunks + NBUF, step, 0, unroll=False)


def kernel(x, token_emb, pos_emb):
    B, T = x.shape
    info = plsc.get_sparse_core_info()
    nw = info.num_cores * info.num_subcores
    total = B * T
    per_w = total // nw
    chunks = per_w // CHUNK
    assert per_w % CHUNK == 0 and per_w % T == 0 and T == 2 * CHUNK
    assert chunks % 2 == 0 and chunks >= NBUF

    x_r = x.astype(jnp.int32).reshape(nw, chunks, CHUNK)
    pos2d = pos_emb[0, :T, :]

    mesh = plsc.VectorSubcoreMesh(core_axis_name="c", subcore_axis_name="s")
    body = functools.partial(_sc_body, num_cores=info.num_cores, chunks=chunks)
    out = pl.kernel(
        body,
        out_type=jax.ShapeDtypeStruct((nw, chunks, CHUNK, EMBED), jnp.float32),
        mesh=mesh,
        compiler_params=pltpu.CompilerParams(use_tc_tiling_on_sc=False),
        scratch_types=[
            pltpu.VMEM((chunks, CHUNK), jnp.int32),
            pltpu.VMEM_SHARED((T, EMBED), jnp.float32),
            pltpu.VMEM((NBUF, CHUNK, EMBED), jnp.float32),
            pltpu.SemaphoreType.DMA((NBUF,)),
            pltpu.SemaphoreType.DMA((NBUF,)),
            pltpu.SemaphoreType.DMA((NBUF,)),
        ],
    )(x_r, token_emb, pos2d)
    return out.reshape(B, T, EMBED)


# speedup vs baseline: 1.5791x; 1.0038x over previous
import functools

import jax
import jax.numpy as jnp
from jax import lax
from jax.experimental import pallas as pl
from jax.experimental.pallas import tpu as pltpu
from jax.experimental.pallas import tpu_sc as plsc

EMBED = 64
CHUNK = 100
NBUF = 6
LOOK = 2


def _sc_body(x_hbm, table_hbm, pos_hbm, out_hbm, idx_v, pos_sh, bufs,
             psem, gsem, osem, *, num_cores, chunks):
    sid = lax.axis_index("s")
    wid = sid * num_cores + lax.axis_index("c")
    pltpu.sync_copy(x_hbm.at[wid], idx_v)

    @pl.when(sid == 0)
    def _():
        for h in range(2):
            sl = pl.ds(h * CHUNK, CHUNK)
            pltpu.sync_copy(pos_hbm.at[sl], bufs.at[0])
            pltpu.sync_copy(bufs.at[0], pos_sh.at[sl])

    plsc.subcore_barrier()

    def step(i, carry):
        b = lax.rem(i, NBUF)
        @pl.when(i >= NBUF)
        def _():
            pltpu.make_async_copy(
                bufs.at[b], out_hbm.at[0, pl.ds(0, CHUNK)], osem.at[b]).wait()

        @pl.when(i < chunks)
        def _():
            pbase = lax.rem(i, 2) * CHUNK
            pltpu.async_copy(
                pos_sh.at[pl.ds(pbase, CHUNK)], bufs.at[b], psem.at[b])

        @pl.when(jnp.logical_and(i >= LOOK, i < chunks + LOOK))
        def _():
            j = i - LOOK
            bj = lax.rem(j, NBUF)
            pltpu.make_async_copy(
                pos_sh.at[pl.ds(0, CHUNK)], bufs.at[bj], psem.at[bj]).wait()
            pltpu.async_copy(
                table_hbm.at[idx_v.at[j]], bufs.at[bj], gsem.at[bj],
                add=True)

        @pl.when(jnp.logical_and(i >= 2 * LOOK, i < chunks + 2 * LOOK))
        def _():
            j = i - 2 * LOOK
            bj = lax.rem(j, NBUF)
            pltpu.make_async_copy(
                table_hbm.at[idx_v.at[j]], bufs.at[bj], gsem.at[bj]).wait()
            b_idx = wid * (chunks // 2) + lax.div(j, 2)
            pltpu.async_copy(
                bufs.at[bj],
                out_hbm.at[b_idx, pl.ds(lax.rem(j, 2) * CHUNK, CHUNK)],
                osem.at[bj])

        return carry

    lax.fori_loop(0, chunks + NBUF, step, 0, unroll=False)


def kernel(x, token_emb, pos_emb):
    B, T = x.shape
    info = plsc.get_sparse_core_info()
    nw = info.num_cores * info.num_subcores
    total = B * T
    per_w = total // nw
    chunks = per_w // CHUNK
    assert per_w % CHUNK == 0 and per_w % T == 0 and T == 2 * CHUNK
    assert chunks % 2 == 0 and chunks >= NBUF

    x_r = x.astype(jnp.int32).reshape(nw, chunks, CHUNK)
    pos2d = pos_emb[0, :T, :]

    mesh = plsc.VectorSubcoreMesh(core_axis_name="c", subcore_axis_name="s")
    body = functools.partial(_sc_body, num_cores=info.num_cores, chunks=chunks)
    out = pl.kernel(
        body,
        out_type=jax.ShapeDtypeStruct((B, T, EMBED), jnp.float32),
        mesh=mesh,
        compiler_params=pltpu.CompilerParams(use_tc_tiling_on_sc=False),
        scratch_types=[
            pltpu.VMEM((chunks, CHUNK), jnp.int32),
            pltpu.VMEM_SHARED((T, EMBED), jnp.float32),
            pltpu.VMEM((NBUF, CHUNK, EMBED), jnp.float32),
            pltpu.SemaphoreType.DMA((NBUF,)),
            pltpu.SemaphoreType.DMA((NBUF,)),
            pltpu.SemaphoreType.DMA((NBUF,)),
        ],
    )(x_r, token_emb, pos2d)
    return out
